# baseline (speedup 1.0000x reference)
import functools
import math

import jax
import jax.numpy as jnp
from jax.experimental import pallas as pl
from jax.experimental.pallas import tpu as pltpu

NH = 12
N = 2048
C = 768
DH = C // NH
KTOP = 1024
SCALE = DH ** -0.5
I16_MIN = -32768
NEG = -3.4028234663852886e38
RCHUNK = 512
CCHUNK = 512


def _proj_kernel(x_ref, w_ref, o_ref):
    o_ref[0] = jax.lax.dot_general(
        x_ref[0], w_ref[0], (((1,), (1,)), ((), ())),
        preferred_element_type=jnp.float32)


def _to_bf16_threshold(cand):
    pat = jnp.where(cand >= 0, cand, cand ^ 0x7FFF)
    pat = jnp.where((cand < 0) & ((pat & 0x7FFF) > 0x7F80), -128, pat)
    return jax.lax.bitcast_convert_type(pat << 16, jnp.float32).astype(
        jnp.bfloat16)


def _count_both(abf_ref, bf_r, bf_c):
    col_parts = []
    cnt_r = jnp.zeros((N, 1), jnp.float32)
    for c0 in range(0, N, CCHUNK):
        blk = abf_ref[:, c0:c0 + CCHUNK]
        cnt_r = cnt_r + jnp.sum((blk >= bf_r).astype(jnp.float32),
                                axis=1, keepdims=True)
        col_parts.append(
            jnp.sum((blk >= bf_c[:, c0:c0 + CCHUNK]).astype(jnp.float32),
                    axis=0, keepdims=True))
    return cnt_r, jnp.concatenate(col_parts, axis=1)


def _search_thresholds(abf_ref):
    kf = jnp.float32(KTOP)
    cnt_r, cnt_c = _count_both(
        abf_ref,
        jnp.zeros((N, 1), jnp.bfloat16), jnp.zeros((1, N), jnp.bfloat16))
    t_r = jnp.where(cnt_r >= kf, jnp.int32(0), jnp.int32(I16_MIN))
    t_c = jnp.where(cnt_c >= kf, jnp.int32(0), jnp.int32(I16_MIN))
    for k in range(14, -1, -1):
        bit = jnp.int32(1 << k)
        cand_r = t_r + bit
        cand_c = t_c + bit
        cnt_r, cnt_c = _count_both(abf_ref, _to_bf16_threshold(cand_r),
                                   _to_bf16_threshold(cand_c))
        t_r = jnp.where(cnt_r >= kf, cand_r, t_r)
        t_c = jnp.where(cnt_c >= kf, cand_c, t_c)
    return _to_bf16_threshold(t_r), _to_bf16_threshold(t_c)


def _attn_kernel(qh_ref, kh_ref, vh_ref, wp_ref, bp_ref, o_ref, abf_ref):
    h = pl.program_id(0)

    @pl.when(h == 0)
    def _init():
        o_ref[...] = jnp.broadcast_to(bp_ref[...], (N, C))

    qb = qh_ref[0] * jnp.float32(SCALE)
    kb = kh_ref[0]

    for r0 in range(0, N, RCHUNK):
        a = jax.lax.dot_general(qb[r0:r0 + RCHUNK], kb,
                                (((1,), (1,)), ((), ())),
                                preferred_element_type=jnp.float32)
        abf_ref[r0:r0 + RCHUNK, :] = a.astype(jnp.bfloat16)

    t_r, t_c = _search_thresholds(abf_ref)

    vb = vh_ref[0]
    for r0 in range(0, N, RCHUNK):
        blk = abf_ref[r0:r0 + RCHUNK, :]
        keep = (blk >= t_r[r0:r0 + RCHUNK]) | (blk >= t_c)
        a = jax.lax.dot_general(qb[r0:r0 + RCHUNK], kb,
                                (((1,), (1,)), ((), ())),
                                preferred_element_type=jnp.float32)
        a = jnp.where(keep, a, NEG)
        mx = jnp.max(a, axis=1, keepdims=True)
        e = jnp.exp(a - mx)
        p = e / jnp.sum(e, axis=1, keepdims=True)
        oh = jax.lax.dot_general(p, vb, (((1,), (0,)), ((), ())),
                                 preferred_element_type=jnp.float32)
        contrib = jax.lax.dot_general(oh, wp_ref[0],
                                      (((1,), (1,)), ((), ())),
                                      preferred_element_type=jnp.float32)
        o_ref[r0:r0 + RCHUNK, :] = o_ref[r0:r0 + RCHUNK, :] + contrib


@functools.partial(jax.jit, static_argnames=())
def kernel(q, k_v, Wq, Wk, Wv, Wp, bp):
    B = q.shape[0]
    q2 = q.reshape(N, C)
    kv2 = k_v.reshape(N, C)

    xs = jnp.stack([q2, kv2, kv2])
    ws = jnp.stack([Wq, Wk, Wv])
    qkv = pl.pallas_call(
        _proj_kernel,
        grid=(3,),
        in_specs=[
            pl.BlockSpec((1, N, C), lambda i: (i, 0, 0)),
            pl.BlockSpec((1, C, C), lambda i: (i, 0, 0)),
        ],
        out_specs=pl.BlockSpec((1, N, C), lambda i: (i, 0, 0)),
        out_shape=jax.ShapeDtypeStruct((3, N, C), jnp.float32),
        compiler_params=pltpu.CompilerParams(
            dimension_semantics=("arbitrary",)),
    )(xs, ws)

    qh = qkv[0].reshape(N, NH, DH).transpose(1, 0, 2)
    kh = qkv[1].reshape(N, NH, DH).transpose(1, 0, 2)
    vh = qkv[2].reshape(N, NH, DH).transpose(1, 0, 2)
    wp3 = Wp.reshape(C, NH, DH).transpose(1, 0, 2)

    out = pl.pallas_call(
        _attn_kernel,
        grid=(NH,),
        in_specs=[
            pl.BlockSpec((1, N, DH), lambda h: (h, 0, 0)),
            pl.BlockSpec((1, N, DH), lambda h: (h, 0, 0)),
            pl.BlockSpec((1, N, DH), lambda h: (h, 0, 0)),
            pl.BlockSpec((1, C, DH), lambda h: (h, 0, 0)),
            pl.BlockSpec((1, C), lambda h: (0, 0)),
        ],
        out_specs=pl.BlockSpec((N, C), lambda h: (0, 0)),
        out_shape=jax.ShapeDtypeStruct((N, C), jnp.float32),
        scratch_shapes=[pltpu.VMEM((N, N), jnp.bfloat16)],
        compiler_params=pltpu.CompilerParams(
            dimension_semantics=("arbitrary",)),
    )(qh, kh, vh, wp3, bp.reshape(1, C))

    return out.reshape(B, N, C)


# speedup vs baseline: 1.5809x; 1.5809x over previous
import functools
import math

import jax
import jax.numpy as jnp
from jax.experimental import pallas as pl
from jax.experimental.pallas import tpu as pltpu

NH = 12
N = 2048
C = 768
DH = C // NH
KTOP = 1024
SCALE = DH ** -0.5
XOR_MASK = 0x7FFFFFFF
INT_MIN = -2147483648
NEG = -3.4028234663852886e38
RCHUNK = 512
CCHUNK = 512
SEARCH_LSB = 18


def _proj_kernel(x_ref, w_ref, o_ref):
    o_ref[0] = jax.lax.dot_general(
        x_ref[0], w_ref[0], (((1,), (1,)), ((), ())),
        preferred_element_type=jnp.float32)


def _count_both(mapped_ref, cand_r, cand_c):
    col_parts = []
    cnt_r = jnp.zeros((N, 1), jnp.float32)
    for c0 in range(0, N, CCHUNK):
        blk = mapped_ref[:, c0:c0 + CCHUNK]
        cnt_r = cnt_r + jnp.sum((blk >= cand_r).astype(jnp.float32),
                                axis=1, keepdims=True)
        col_parts.append(
            jnp.sum((blk >= cand_c[:, c0:c0 + CCHUNK]).astype(jnp.float32),
                    axis=0, keepdims=True))
    return cnt_r, jnp.concatenate(col_parts, axis=1)


def _search_thresholds(mapped_ref):
    kf = jnp.float32(KTOP)
    cnt_r, cnt_c = _count_both(mapped_ref, jnp.zeros((N, 1), jnp.int32),
                               jnp.zeros((1, N), jnp.int32))
    t_r = jnp.where(cnt_r >= kf, jnp.int32(0), INT_MIN)
    t_c = jnp.where(cnt_c >= kf, jnp.int32(0), INT_MIN)
    for k in range(30, SEARCH_LSB - 1, -1):
        bit = jnp.int32(1 << k)
        cand_r = t_r + bit
        cand_c = t_c + bit
        cnt_r, cnt_c = _count_both(mapped_ref, cand_r, cand_c)
        t_r = jnp.where(cnt_r >= kf, cand_r, t_r)
        t_c = jnp.where(cnt_c >= kf, cand_c, t_c)
    return t_r, t_c


def _attn_kernel(qh_ref, kh_ref, vh_ref, wp_ref, bp_ref, o_ref, mapped_ref):
    h = pl.program_id(0)

    @pl.when(h == 0)
    def _init():
        o_ref[...] = jnp.broadcast_to(bp_ref[...], (N, C))

    qb = qh_ref[0] * jnp.float32(SCALE)
    kb = kh_ref[0]

    for r0 in range(0, N, RCHUNK):
        a = jax.lax.dot_general(qb[r0:r0 + RCHUNK], kb,
                                (((1,), (1,)), ((), ())),
                                preferred_element_type=jnp.float32)
        bits = jax.lax.bitcast_convert_type(a, jnp.int32)
        mapped_ref[r0:r0 + RCHUNK, :] = jnp.where(bits >= 0, bits,
                                                  bits ^ XOR_MASK)

    t_row, t_col = _search_thresholds(mapped_ref)

    vb = vh_ref[0]
    for r0 in range(0, N, RCHUNK):
        m = mapped_ref[r0:r0 + RCHUNK, :]
        keep = (m >= t_row[r0:r0 + RCHUNK]) | (m >= t_col)
        a = jax.lax.bitcast_convert_type(jnp.where(m >= 0, m, m ^ XOR_MASK),
                                         jnp.float32)
        a = jnp.where(keep, a, NEG)
        mx = jnp.max(a, axis=1, keepdims=True)
        e = jnp.exp(a - mx)
        p = e / jnp.sum(e, axis=1, keepdims=True)
        oh = jax.lax.dot_general(p, vb, (((1,), (0,)), ((), ())),
                                 preferred_element_type=jnp.float32)
        contrib = jax.lax.dot_general(oh, wp_ref[0],
                                      (((1,), (1,)), ((), ())),
                                      preferred_element_type=jnp.float32)
        o_ref[r0:r0 + RCHUNK, :] = o_ref[r0:r0 + RCHUNK, :] + contrib


@functools.partial(jax.jit, static_argnames=())
def kernel(q, k_v, Wq, Wk, Wv, Wp, bp):
    B = q.shape[0]
    q2 = q.reshape(N, C)
    kv2 = k_v.reshape(N, C)

    xs = jnp.stack([q2, kv2, kv2])
    ws = jnp.stack([Wq, Wk, Wv])
    qkv = pl.pallas_call(
        _proj_kernel,
        grid=(3,),
        in_specs=[
            pl.BlockSpec((1, N, C), lambda i: (i, 0, 0)),
            pl.BlockSpec((1, C, C), lambda i: (i, 0, 0)),
        ],
        out_specs=pl.BlockSpec((1, N, C), lambda i: (i, 0, 0)),
        out_shape=jax.ShapeDtypeStruct((3, N, C), jnp.float32),
        compiler_params=pltpu.CompilerParams(
            dimension_semantics=("arbitrary",)),
    )(xs, ws)

    qh = qkv[0].reshape(N, NH, DH).transpose(1, 0, 2)
    kh = qkv[1].reshape(N, NH, DH).transpose(1, 0, 2)
    vh = qkv[2].reshape(N, NH, DH).transpose(1, 0, 2)
    wp3 = Wp.reshape(C, NH, DH).transpose(1, 0, 2)

    out = pl.pallas_call(
        _attn_kernel,
        grid=(NH,),
        in_specs=[
            pl.BlockSpec((1, N, DH), lambda h: (h, 0, 0)),
            pl.BlockSpec((1, N, DH), lambda h: (h, 0, 0)),
            pl.BlockSpec((1, N, DH), lambda h: (h, 0, 0)),
            pl.BlockSpec((1, C, DH), lambda h: (h, 0, 0)),
            pl.BlockSpec((1, C), lambda h: (0, 0)),
        ],
        out_specs=pl.BlockSpec((N, C), lambda h: (0, 0)),
        out_shape=jax.ShapeDtypeStruct((N, C), jnp.float32),
        scratch_shapes=[pltpu.VMEM((N, N), jnp.int32)],
        compiler_params=pltpu.CompilerParams(
            dimension_semantics=("arbitrary",)),
    )(qh, kh, vh, wp3, bp.reshape(1, C))

    return out.reshape(B, N, C)


# speedup vs baseline: 1.5810x; 1.0001x over previous
import functools
import math

import jax
import jax.numpy as jnp
from jax.experimental import pallas as pl
from jax.experimental.pallas import tpu as pltpu

NH = 12
N = 2048
C = 768
DH = C // NH
KTOP = 1024
SCALE = DH ** -0.5
XOR_MASK = 0x7FFFFFFF
INT_MIN = -2147483648
NEG = -3.4028234663852886e38
RCHUNK = 512
CCHUNK = 1024
SEARCH_LSB = 18


def _proj_kernel(x_ref, w_ref, o_ref):
    o_ref[0] = jax.lax.dot_general(
        x_ref[0], w_ref[0], (((1,), (1,)), ((), ())),
        preferred_element_type=jnp.float32)


def _count_both(mapped_ref, cand_r, cand_c):
    col_parts = []
    cnt_r = jnp.zeros((N, 1), jnp.float32)
    for c0 in range(0, N, CCHUNK):
        blk = mapped_ref[:, c0:c0 + CCHUNK]
        cnt_r = cnt_r + jnp.sum((blk >= cand_r).astype(jnp.float32),
                                axis=1, keepdims=True)
        col_parts.append(
            jnp.sum((blk >= cand_c[:, c0:c0 + CCHUNK]).astype(jnp.float32),
                    axis=0, keepdims=True))
    return cnt_r, jnp.concatenate(col_parts, axis=1)


def _search_thresholds(mapped_ref):
    kf = jnp.float32(KTOP)
    cnt_r, cnt_c = _count_both(mapped_ref, jnp.zeros((N, 1), jnp.int32),
                               jnp.zeros((1, N), jnp.int32))
    t_r = jnp.where(cnt_r >= kf, jnp.int32(0), INT_MIN)
    t_c = jnp.where(cnt_c >= kf, jnp.int32(0), INT_MIN)
    for k in range(30, SEARCH_LSB - 1, -1):
        bit = jnp.int32(1 << k)
        cand_r = t_r + bit
        cand_c = t_c + bit
        cnt_r, cnt_c = _count_both(mapped_ref, cand_r, cand_c)
        t_r = jnp.where(cnt_r >= kf, cand_r, t_r)
        t_c = jnp.where(cnt_c >= kf, cand_c, t_c)
    return t_r, t_c


def _attn_kernel(qh_ref, kh_ref, vh_ref, wp_ref, bp_ref, o_ref, mapped_ref):
    h = pl.program_id(0)

    @pl.when(h == 0)
    def _init():
        o_ref[...] = jnp.broadcast_to(bp_ref[...], (N, C))

    qb = qh_ref[0] * jnp.float32(SCALE)
    kb = kh_ref[0]

    for r0 in range(0, N, RCHUNK):
        a = jax.lax.dot_general(qb[r0:r0 + RCHUNK], kb,
                                (((1,), (1,)), ((), ())),
                                preferred_element_type=jnp.float32)
        bits = jax.lax.bitcast_convert_type(a, jnp.int32)
        mapped_ref[r0:r0 + RCHUNK, :] = jnp.where(bits >= 0, bits,
                                                  bits ^ XOR_MASK)

    t_row, t_col = _search_thresholds(mapped_ref)

    vb = vh_ref[0]
    for r0 in range(0, N, RCHUNK):
        m = mapped_ref[r0:r0 + RCHUNK, :]
        keep = (m >= t_row[r0:r0 + RCHUNK]) | (m >= t_col)
        a = jax.lax.bitcast_convert_type(jnp.where(m >= 0, m, m ^ XOR_MASK),
                                         jnp.float32)
        a = jnp.where(keep, a, NEG)
        mx = jnp.max(a, axis=1, keepdims=True)
        ex = jnp.exp(a - mx)
        p = ex / jnp.sum(ex, axis=1, keepdims=True)
        oh = jax.lax.dot_general(p, vb, (((1,), (0,)), ((), ())),
                                 preferred_element_type=jnp.float32)
        contrib = jax.lax.dot_general(oh, wp_ref[0],
                                      (((1,), (1,)), ((), ())),
                                      preferred_element_type=jnp.float32)
        o_ref[r0:r0 + RCHUNK, :] = o_ref[r0:r0 + RCHUNK, :] + contrib


@functools.partial(jax.jit, static_argnames=())
def kernel(q, k_v, Wq, Wk, Wv, Wp, bp):
    B = q.shape[0]
    q2 = q.reshape(N, C)
    kv2 = k_v.reshape(N, C)

    xs = jnp.stack([q2, kv2, kv2])
    ws = jnp.stack([Wq, Wk, Wv])
    qkv = pl.pallas_call(
        _proj_kernel,
        grid=(3,),
        in_specs=[
            pl.BlockSpec((1, N, C), lambda i: (i, 0, 0)),
            pl.BlockSpec((1, C, C), lambda i: (i, 0, 0)),
        ],
        out_specs=pl.BlockSpec((1, N, C), lambda i: (i, 0, 0)),
        out_shape=jax.ShapeDtypeStruct((3, N, C), jnp.float32),
        compiler_params=pltpu.CompilerParams(
            dimension_semantics=("arbitrary",)),
    )(xs, ws)

    qh = qkv[0].reshape(N, NH, DH).transpose(1, 0, 2)
    kh = qkv[1].reshape(N, NH, DH).transpose(1, 0, 2)
    vh = qkv[2].reshape(N, NH, DH).transpose(1, 0, 2)
    wp3 = Wp.reshape(C, NH, DH).transpose(1, 0, 2)

    out = pl.pallas_call(
        _attn_kernel,
        grid=(NH,),
        in_specs=[
            pl.BlockSpec((1, N, DH), lambda h: (h, 0, 0)),
            pl.BlockSpec((1, N, DH), lambda h: (h, 0, 0)),
            pl.BlockSpec((1, N, DH), lambda h: (h, 0, 0)),
            pl.BlockSpec((1, C, DH), lambda h: (h, 0, 0)),
            pl.BlockSpec((1, C), lambda h: (0, 0)),
        ],
        out_specs=pl.BlockSpec((N, C), lambda h: (0, 0)),
        out_shape=jax.ShapeDtypeStruct((N, C), jnp.float32),
        scratch_shapes=[pltpu.VMEM((N, N), jnp.int32)],
        compiler_params=pltpu.CompilerParams(
            dimension_semantics=("arbitrary",)),
    )(qh, kh, vh, wp3, bp.reshape(1, C))

    return out.reshape(B, N, C)
